# baseline (speedup 1.0000x reference)
import functools

import jax
import jax.numpy as jnp
from jax import lax
from jax.experimental import pallas as pl
from jax.experimental.pallas import tpu as pltpu
from jax.experimental.pallas import tpu_sc as plsc

NC = 2
NS = 16
LANES = 16
CHUNK = 128


@functools.cache
def _build(B, N, D):
    NW = NC * NS
    b_per_w = B // NW
    n_chunks = b_per_w // CHUNK
    n_groups = b_per_w // LANES

    mesh = plsc.VectorSubcoreMesh(
        core_axis_name="c", subcore_axis_name="s",
        num_cores=NC, num_subcores=NS,
    )

    @functools.partial(
        pl.kernel,
        out_type=jax.ShapeDtypeStruct((B,), jnp.float32),
        mesh=mesh,
        compiler_params=pltpu.CompilerParams(
            needs_layout_passes=False, use_tc_tiling_on_sc=False),
        scratch_types=[
            pltpu.VMEM((n_chunks, CHUNK), jnp.int32),
            pltpu.VMEM((n_chunks, CHUNK), jnp.int32),
            pltpu.VMEM((n_chunks, CHUNK), jnp.int32),
            pltpu.VMEM((b_per_w, D), jnp.float32),
            pltpu.VMEM((b_per_w, D), jnp.float32),
            pltpu.VMEM((b_per_w, D), jnp.float32),
            pltpu.VMEM((b_per_w,), jnp.float32),
            pltpu.SemaphoreType.DMA,
        ],
    )
    def kern(i_hbm, j_hbm, k_hbm, u_hbm, v_hbm, out_hbm,
             iv, jv, kv, ur, vjr, vkr, outv, sem):
        wid = lax.axis_index("s") * NC + lax.axis_index("c")
        blk0 = wid * n_chunks

        pltpu.sync_copy(i_hbm.at[pl.ds(blk0, n_chunks)], iv)
        pltpu.sync_copy(j_hbm.at[pl.ds(blk0, n_chunks)], jv)
        pltpu.sync_copy(k_hbm.at[pl.ds(blk0, n_chunks)], kv)

        copies = []
        for c in range(n_chunks):
            dst = pl.ds(c * CHUNK, CHUNK)
            copies.append(pltpu.async_copy(u_hbm.at[iv.at[c]], ur.at[dst], sem))
            copies.append(pltpu.async_copy(v_hbm.at[jv.at[c]], vjr.at[dst], sem))
            copies.append(pltpu.async_copy(v_hbm.at[kv.at[c]], vkr.at[dst], sem))
        for cp in copies:
            cp.wait()

        lane = lax.iota(jnp.int32, LANES)

        def group_body(g, _):
            rid = lane + g * LANES
            acc = jnp.zeros((LANES,), jnp.float32)
            for d in range(D):
                col = jnp.full((LANES,), d, jnp.int32)
                u_d = plsc.load_gather(ur, [rid, col])
                vj_d = plsc.load_gather(vjr, [rid, col])
                vk_d = plsc.load_gather(vkr, [rid, col])
                dj = u_d - vj_d
                dk = u_d - vk_d
                acc = acc + (dk * dk - dj * dj)
            outv[pl.ds(g * LANES, LANES)] = 1.0 / (1.0 + jnp.exp(-acc))
            return _

        lax.fori_loop(0, n_groups, group_body, None)
        pltpu.sync_copy(outv, out_hbm.at[pl.ds(wid * b_per_w, b_per_w)])

    return kern


def kernel(i, j, k, u_weight, v_weight):
    B = i.shape[0]
    N, D = u_weight.shape
    kern = _build(B, N, D)
    i2 = i.astype(jnp.int32).reshape(B // CHUNK, CHUNK)
    j2 = j.astype(jnp.int32).reshape(B // CHUNK, CHUNK)
    k2 = k.astype(jnp.int32).reshape(B // CHUNK, CHUNK)
    return kern(i2, j2, k2, u_weight, v_weight)


# speedup vs baseline: 1.2461x; 1.2461x over previous
import functools

import jax
import jax.numpy as jnp
from jax import lax
from jax.experimental import pallas as pl
from jax.experimental.pallas import tpu as pltpu
from jax.experimental.pallas import tpu_sc as plsc

NC = 2
NS = 16
LANES = 16


CHUNK = 128


@functools.cache
def _build(B, N, D):
    NW = NC * NS
    b_per_w = B // NW
    n_chunks = b_per_w // CHUNK
    groups_per_chunk = CHUNK // LANES

    mesh = plsc.VectorSubcoreMesh(
        core_axis_name="c", subcore_axis_name="s",
        num_cores=NC, num_subcores=NS,
    )

    @functools.partial(
        pl.kernel,
        out_type=jax.ShapeDtypeStruct((B,), jnp.float32),
        mesh=mesh,
        compiler_params=pltpu.CompilerParams(
            needs_layout_passes=False, use_tc_tiling_on_sc=True),
        scratch_types=[
            pltpu.VMEM((b_per_w,), jnp.int32),
            pltpu.VMEM((b_per_w,), jnp.int32),
            pltpu.VMEM((b_per_w,), jnp.int32),
            pltpu.VMEM((CHUNK, D), jnp.float32),
            pltpu.VMEM((CHUNK, D), jnp.float32),
            pltpu.VMEM((CHUNK, D), jnp.float32),
            pltpu.VMEM((b_per_w,), jnp.float32),
            pltpu.SemaphoreType.DMA,
        ],
    )
    def kern(i_hbm, j_hbm, k_hbm, u_hbm, v_hbm, out_hbm,
             iv, jv, kv, ur, vjr, vkr, outv, sem):
        wid = lax.axis_index("s") * NC + lax.axis_index("c")
        base = wid * b_per_w

        pltpu.sync_copy(i_hbm.at[pl.ds(base, b_per_w)], iv)
        pltpu.sync_copy(j_hbm.at[pl.ds(base, b_per_w)], jv)
        pltpu.sync_copy(k_hbm.at[pl.ds(base, b_per_w)], kv)

        def issue_group(co, g):
            ivec = iv[pl.ds(co + g * LANES, LANES)]
            jvec = jv[pl.ds(co + g * LANES, LANES)]
            kvec = kv[pl.ds(co + g * LANES, LANES)]
            for l in range(LANES):
                row = g * LANES + l
                pltpu.async_copy(u_hbm.at[ivec[l]], ur.at[row], sem)
                pltpu.async_copy(v_hbm.at[jvec[l]], vjr.at[row], sem)
                pltpu.async_copy(v_hbm.at[kvec[l]], vkr.at[row], sem)

        def drain_groups(n):
            slab = pl.ds(0, LANES * n)
            pltpu.make_async_copy(u_hbm.at[slab], ur.at[slab], sem).wait()
            pltpu.make_async_copy(v_hbm.at[slab], vjr.at[slab], sem).wait()
            pltpu.make_async_copy(v_hbm.at[slab], vkr.at[slab], sem).wait()

        lane = lax.iota(jnp.int32, LANES)

        def compute_group(co, g):
            rid = lane + g * LANES
            acc = jnp.zeros((LANES,), jnp.float32)
            for d in range(D):
                col = jnp.full((LANES,), d, jnp.int32)
                u_d = plsc.load_gather(ur, [rid, col])
                vj_d = plsc.load_gather(vjr, [rid, col])
                vk_d = plsc.load_gather(vkr, [rid, col])
                dj = u_d - vj_d
                dk = u_d - vk_d
                acc = acc + (dk * dk - dj * dj)
            outv[pl.ds(co + g * LANES, LANES)] = 1.0 / (1.0 + jnp.exp(-acc))

        def chunk_body(c, _):
            co = c * CHUNK

            def dma_body(g, _):
                issue_group(co, g)
                return _

            lax.fori_loop(0, groups_per_chunk, dma_body, None)
            drain_groups(groups_per_chunk)

            def comp_body(g, _):
                compute_group(co, g)
                return _

            lax.fori_loop(0, groups_per_chunk, comp_body, None)
            return _

        lax.fori_loop(0, n_chunks, chunk_body, None)
        pltpu.sync_copy(outv, out_hbm.at[pl.ds(base, b_per_w)])

    return kern


def kernel(i, j, k, u_weight, v_weight):
    B = i.shape[0]
    N, D = u_weight.shape
    kern = _build(B, N, D)
    return kern(i.astype(jnp.int32), j.astype(jnp.int32), k.astype(jnp.int32),
                u_weight, v_weight)
